# baseline (speedup 1.0000x reference)
import functools

import jax
import jax.numpy as jnp
from jax import lax
from jax.experimental import pallas as pl
from jax.experimental.pallas import tpu as pltpu
from jax.experimental.pallas import tpu_sc as plsc

NC, NS = 2, 16
NW = NC * NS
EMSIZE = 64
B, L = 1024, 200
BT = B // 128
LT = L // 8
NL = (L + NW - 1) // NW


@jax.jit
def _sc_gather(idx4, table):
    mesh = plsc.VectorSubcoreMesh(
        core_axis_name="c", subcore_axis_name="s", num_cores=NC, num_subcores=NS)

    @functools.partial(
        pl.kernel,
        out_type=jax.ShapeDtypeStruct((L, 8, BT, 8, 128), jnp.float32),
        mesh=mesh,
        scratch_types=[
            pltpu.VMEM((BT, 128), jnp.int32),
            pltpu.VMEM((3, 128, EMSIZE), jnp.float32),
            pltpu.VMEM((3, 8, 8, 128), jnp.float32),
            [pltpu.SemaphoreType.DMA] * 3,
            [pltpu.SemaphoreType.DMA] * 3,
        ],
        compiler_params=pltpu.CompilerParams(
            use_tc_tiling_on_sc=False, needs_layout_passes=False),
    )
    def k(idx_hbm, table_hbm, out_hbm, idx_v, bufa, bufb, gsems, wsems):
        wid = lax.axis_index("s") * NC + lax.axis_index("c")

        row16 = jnp.arange(16, dtype=jnp.int32)
        rowpats = [row16 + (b0 * 16) for b0 in range(8)]
        colbase = jnp.zeros((16,), dtype=jnp.int32)

        def fire_gather(bt, s):
            pltpu.async_copy(
                table_hbm.at[idx_v.at[bt]], bufa.at[s], gsems[s])

        def wait_gather(s):
            pltpu.make_async_copy(
                table_hbm.at[pl.ds(0, 128)], bufa.at[s], gsems[s]).wait()

        def transpose(s):
            def body(e, carry):
                ed, em = e // 8, e % 8
                col = colbase + e
                for b0 in range(8):
                    v = plsc.load_gather(bufa.at[s], [rowpats[b0], col])
                    bufb[s, ed, em, pl.ds(b0 * 16, 16)] = v
                return carry
            lax.fori_loop(0, EMSIZE, body, 0)

        def fire_write(l, bt, s):
            pltpu.async_copy(bufb.at[s], out_hbm.at[l, :, bt], wsems[s])

        def wait_write(s):
            pltpu.make_async_copy(
                out_hbm.at[0, :, 0], bufb.at[s], wsems[s]).wait()

        def do_l(l):
            pltpu.sync_copy(idx_hbm.at[l // 8, :, l % 8], idx_v)
            for bt in range(3):
                fire_gather(bt, bt)
            for bt in range(8):
                s = bt % 3
                wait_gather(s)
                if bt >= 3:
                    wait_write(s)
                transpose(s)
                fire_write(l, bt, s)
                if bt + 3 < 8:
                    fire_gather(bt + 3, s)
            for bt in range(5, 8):
                wait_write(bt % 3)

        def lbody(t, carry):
            l = t * NW + wid

            @pl.when(l < L)
            def _():
                do_l(l)
            return carry

        lax.fori_loop(0, NL, lbody, 0)

    return k(idx4, table)


def kernel(sequence, sequence_char, src_word_table):
    idx4 = sequence.reshape(BT, 128, LT, 8).transpose(2, 0, 3, 1)
    out5 = _sc_gather(idx4, src_word_table)
    return out5.transpose(2, 4, 0, 1, 3).reshape(B, L, EMSIZE)


# speedup vs baseline: 1.3611x; 1.3611x over previous
import functools

import jax
import jax.numpy as jnp
from jax import lax
from jax.experimental import pallas as pl
from jax.experimental.pallas import tpu as pltpu
from jax.experimental.pallas import tpu_sc as plsc

NC, NS = 2, 16
NW = NC * NS
EMSIZE = 64
B, L = 1024, 200
BT = B // 128
LT = L // 8
NL = (L + NW - 1) // NW
VOCAB_N = 1000000
CBLK = 2048
HBLK = (VOCAB_N // 2 + CBLK - 1) // CBLK + 1
P2 = HBLK * CBLK


@jax.jit
def _tc_detile(table_t):
    def body(lo_ref, hi_ref, out_ref):
        out_ref[...] = jnp.concatenate(
            [lo_ref[...].T, hi_ref[...].T], axis=1)

    return pl.pallas_call(
        body,
        grid=(HBLK,),
        in_specs=[
            pl.BlockSpec((EMSIZE, CBLK), lambda i: (0, i)),
            pl.BlockSpec(
                (EMSIZE, CBLK),
                lambda i: (0, jnp.minimum(HBLK + i, VOCAB_N // CBLK))),
        ],
        out_specs=pl.BlockSpec((CBLK, 128), lambda i: (i, 0)),
        out_shape=jax.ShapeDtypeStruct((P2, 128), jnp.float32),
    )(table_t, table_t)


@jax.jit
def _sc_gather(idx4, packed):
    mesh = plsc.VectorSubcoreMesh(
        core_axis_name="c", subcore_axis_name="s", num_cores=NC, num_subcores=NS)

    @functools.partial(
        pl.kernel,
        out_type=jax.ShapeDtypeStruct((L, 8, BT, 8, 128), jnp.float32),
        mesh=mesh,
        scratch_types=[
            pltpu.VMEM((BT, 128), jnp.int32),
            pltpu.VMEM((3, 128), jnp.int32),
            pltpu.VMEM((3, 128, 128), jnp.float32),
            pltpu.VMEM((3, 8, 8, 128), jnp.float32),
            [pltpu.SemaphoreType.DMA] * 3,
            [pltpu.SemaphoreType.DMA] * 3,
        ],
        compiler_params=pltpu.CompilerParams(
            use_tc_tiling_on_sc=False, needs_layout_passes=False),
    )
    def k(idx_hbm, tbl_hbm, out_hbm, idx_v, pidx, bufa, bufb, gsems, wsems):
        wid = lax.axis_index("s") * NC + lax.axis_index("c")

        row16 = jnp.arange(16, dtype=jnp.int32)
        rowpats = [row16 + (b0 * 16) for b0 in range(8)]

        def fire_gather(bt, s):
            for b0 in range(8):
                r = idx_v[bt, pl.ds(b0 * 16, 16)]
                hi = jnp.where(r >= P2, P2, 0)
                pidx[s, pl.ds(b0 * 16, 16)] = r - hi
            pltpu.async_copy(tbl_hbm.at[pidx.at[s]], bufa.at[s], gsems[s])

        def wait_gather(s):
            pltpu.make_async_copy(
                tbl_hbm.at[pl.ds(0, 128)], bufa.at[s], gsems[s]).wait()

        def transpose(bt, s):
            pars = [
                jnp.where(idx_v[bt, pl.ds(b0 * 16, 16)] >= P2, EMSIZE, 0)
                for b0 in range(8)
            ]

            def body(e, carry):
                ed, em = e // 8, e % 8
                for b0 in range(8):
                    v = plsc.load_gather(bufa.at[s], [rowpats[b0], pars[b0] + e])
                    bufb[s, ed, em, pl.ds(b0 * 16, 16)] = v
                return carry
            lax.fori_loop(0, EMSIZE, body, 0)

        def fire_write(l, bt, s):
            pltpu.async_copy(bufb.at[s], out_hbm.at[l, :, bt], wsems[s])

        def wait_write(s):
            pltpu.make_async_copy(
                out_hbm.at[0, :, 0], bufb.at[s], wsems[s]).wait()

        def do_l(l):
            pltpu.sync_copy(idx_hbm.at[l // 8, :, l % 8], idx_v)
            for bt in range(3):
                fire_gather(bt, bt)
            for bt in range(8):
                s = bt % 3
                wait_gather(s)
                if bt >= 3:
                    wait_write(s)
                transpose(bt, s)
                fire_write(l, bt, s)
                if bt + 3 < 8:
                    fire_gather(bt + 3, s)
            for bt in range(5, 8):
                wait_write(bt % 3)

        def lbody(t, carry):
            l = t * NW + wid

            @pl.when(l < L)
            def _():
                do_l(l)
            return carry

        lax.fori_loop(0, NL, lbody, 0)

    return k(idx4, packed)


def kernel(sequence, sequence_char, src_word_table):
    packed = _tc_detile(src_word_table.T)
    idx4 = sequence.reshape(BT, 128, LT, 8).transpose(2, 0, 3, 1)
    out5 = _sc_gather(idx4, packed)
    return out5.transpose(2, 4, 0, 1, 3).reshape(B, L, EMSIZE)


# speedup vs baseline: 1.3812x; 1.0148x over previous
import functools

import jax
import jax.numpy as jnp
from jax import lax
from jax.experimental import pallas as pl
from jax.experimental.pallas import tpu as pltpu
from jax.experimental.pallas import tpu_sc as plsc

NC, NS = 2, 16
NW = NC * NS
EMSIZE = 64
B, L = 1024, 200
BT = B // 128
LT = L // 8
NL = (L + NW - 1) // NW
VOCAB_N = 1000000
CBLK = 2048
HBLK = (VOCAB_N // 2 + CBLK - 1) // CBLK + 1
P2 = HBLK * CBLK


@jax.jit
def _tc_detile(table_t):
    def body(lo_ref, hi_ref, out_ref):
        out_ref[...] = jnp.concatenate(
            [lo_ref[...].T, hi_ref[...].T], axis=1)

    return pl.pallas_call(
        body,
        grid=(HBLK,),
        in_specs=[
            pl.BlockSpec((EMSIZE, CBLK), lambda i: (0, i)),
            pl.BlockSpec(
                (EMSIZE, CBLK),
                lambda i: (0, jnp.minimum(HBLK + i, VOCAB_N // CBLK))),
        ],
        out_specs=pl.BlockSpec((CBLK, 128), lambda i: (i, 0)),
        out_shape=jax.ShapeDtypeStruct((P2, 128), jnp.float32),
    )(table_t, table_t)


@jax.jit
def _sc_gather(idx4, packed):
    mesh = plsc.VectorSubcoreMesh(
        core_axis_name="c", subcore_axis_name="s", num_cores=NC, num_subcores=NS)

    @functools.partial(
        pl.kernel,
        out_type=jax.ShapeDtypeStruct((L, 8, BT, 8, 128), jnp.float32),
        mesh=mesh,
        scratch_types=[
            pltpu.VMEM((BT, 128), jnp.int32),
            pltpu.VMEM((4, 128), jnp.int32),
            pltpu.VMEM((4, 128, EMSIZE), jnp.float32),
            pltpu.VMEM((4, 8, 8, 128), jnp.float32),
            [pltpu.SemaphoreType.DMA] * 4,
            [pltpu.SemaphoreType.DMA] * 4,
        ],
        compiler_params=pltpu.CompilerParams(
            use_tc_tiling_on_sc=False, needs_layout_passes=False),
    )
    def k(idx_hbm, tbl_hbm, out_hbm, idx_v, pidx, bufa, bufb, gsems, wsems):
        wid = lax.axis_index("s") * NC + lax.axis_index("c")

        row16 = jnp.arange(16, dtype=jnp.int32)
        rowpats = [row16 + (b0 * 16) for b0 in range(8)]
        colz = jnp.zeros((16,), dtype=jnp.int32)

        def fire_gather(bt, s):
            for b0 in range(8):
                r = idx_v[bt, pl.ds(b0 * 16, 16)]
                hi = jnp.where(r >= P2, 1, 0)
                pidx[s, pl.ds(b0 * 16, 16)] = 2 * (r - hi * P2) + hi
            pltpu.async_copy(tbl_hbm.at[pidx.at[s]], bufa.at[s], gsems[s])

        def wait_gather(s):
            pltpu.make_async_copy(
                tbl_hbm.at[pl.ds(0, 128)], bufa.at[s], gsems[s]).wait()

        def transpose(bt, s):
            def body(e, carry):
                ed, em = e // 8, e % 8
                for b0 in range(8):
                    v = plsc.load_gather(bufa.at[s], [rowpats[b0], colz + e])
                    bufb[s, ed, em, pl.ds(b0 * 16, 16)] = v
                return carry
            lax.fori_loop(0, EMSIZE, body, 0)

        def fire_write(l, bt, s):
            pltpu.async_copy(bufb.at[s], out_hbm.at[l, :, bt], wsems[s])

        def wait_write(s):
            pltpu.make_async_copy(
                out_hbm.at[0, :, 0], bufb.at[s], wsems[s]).wait()

        def do_l(l):
            pltpu.sync_copy(idx_hbm.at[l // 8, :, l % 8], idx_v)
            for bt in range(4):
                fire_gather(bt, bt)
            for bt in range(8):
                s = bt % 4
                wait_gather(s)
                if bt >= 4:
                    wait_write(s)
                transpose(bt, s)
                fire_write(l, bt, s)
                if bt + 4 < 8:
                    fire_gather(bt + 4, s)
            for bt in range(4, 8):
                wait_write(bt % 4)

        def lbody(t, carry):
            l = t * NW + wid

            @pl.when(l < L)
            def _():
                do_l(l)
            return carry

        lax.fori_loop(0, NL, lbody, 0)

    return k(idx4, packed)


def kernel(sequence, sequence_char, src_word_table):
    packed = _tc_detile(src_word_table.T).reshape(2 * P2, EMSIZE)
    idx4 = sequence.reshape(BT, 128, LT, 8).transpose(2, 0, 3, 1)
    out5 = _sc_gather(idx4, packed)
    return out5.transpose(2, 4, 0, 1, 3).reshape(B, L, EMSIZE)
